# Initial kernel scaffold; baseline (speedup 1.0000x reference)
import jax
import jax.numpy as jnp
from jax.experimental import pallas as pl


def kernel(emb, W_sm, b_sm, coeff1, bases1, bias1, coeff2, bases2, bias2, norm, node_feats, edge_index, etypes):
    raise NotImplementedError("write your pallas kernel here")


# speedup vs baseline: 8.9143x; 8.9143x over previous
import functools

import jax
import jax.numpy as jnp
from jax import lax
from jax.experimental import pallas as pl
from jax.experimental.pallas import tpu as pltpu
from jax.experimental.pallas import tpu_sc as plsc

N = 10000
E = 320000
R = 8
NB = 8
H = 128
EDIM = 16
ML = 8
VOCAB = 4
OUT = 128

NC = 2
NS = 16
NW = NC * NS
EPW = E // NW
CHUNK = 80
NCHUNK = EPW // CHUNK
ZROWS = 624
ZREM = N - NS * ZROWS

BLK = 1000
NBLK = N // BLK

_HIGH = jax.lax.Precision.HIGHEST


def _sc_body(table, src_h, dst_h, et_h, nrm_h, out_h,
             src_v, dst_v, et_v, nrm_v, gidx_v, rows_v, acc):
    c = lax.axis_index("c")
    s = lax.axis_index("s")
    wid = s * NC + c
    base = wid * EPW

    zeros16 = jnp.zeros((16,), jnp.float32)

    @pl.loop(0, CHUNK)
    def _zero_rows(i):
        for f in range(H // 16):
            rows_v[i, pl.ds(f * 16, 16)] = zeros16

    zbase = s * ZROWS
    nfull = ZROWS // CHUNK
    tail = ZROWS - nfull * CHUNK

    @pl.loop(0, nfull)
    def _zero_acc(j):
        pltpu.sync_copy(rows_v, acc.at[pl.ds(zbase + j * CHUNK, CHUNK)])

    pltpu.sync_copy(rows_v.at[pl.ds(0, tail)],
                    acc.at[pl.ds(zbase + nfull * CHUNK, tail)])

    @pl.when(s == NS - 1)
    def _zero_rem():
        pltpu.sync_copy(rows_v.at[pl.ds(0, ZREM)],
                        acc.at[pl.ds(NS * ZROWS, ZREM)])

    plsc.subcore_barrier()

    @pl.loop(0, NCHUNK)
    def _chunk(i):
        o = base + i * CHUNK
        pltpu.sync_copy(src_h.at[pl.ds(o, CHUNK)], src_v)
        pltpu.sync_copy(et_h.at[pl.ds(o, CHUNK)], et_v)
        pltpu.sync_copy(dst_h.at[pl.ds(o, CHUNK)], dst_v)
        pltpu.sync_copy(nrm_h.at[pl.ds(o, CHUNK)], nrm_v)

        for g in range(CHUNK // 16):
            sl = pl.ds(g * 16, 16)
            gidx_v[sl] = et_v[sl] * N + src_v[sl]

        pltpu.sync_copy(table.at[gidx_v], rows_v)

        @pl.loop(0, CHUNK)
        def _scale(e):
            nb = plsc.load_gather(nrm_v, [lax.broadcast(e, (16,))])
            for f in range(H // 16):
                sl = pl.ds(f * 16, 16)
                rows_v[e, sl] = rows_v[e, sl] * nb

        pltpu.sync_copy(rows_v, acc.at[dst_v], add=True)

    plsc.subcore_barrier()

    pltpu.sync_copy(acc.at[pl.ds(zbase, ZROWS)],
                    out_h.at[c, pl.ds(zbase, ZROWS)])

    @pl.when(s == NS - 1)
    def _wb_rem():
        pltpu.sync_copy(acc.at[pl.ds(NS * ZROWS, ZREM)],
                        out_h.at[c, pl.ds(NS * ZROWS, ZREM)])


def _sc_aggregate(table, src, dst, etypes, nrm):
    mesh = plsc.VectorSubcoreMesh(core_axis_name="c", subcore_axis_name="s",
                                  num_cores=NC, num_subcores=NS)
    fn = pl.kernel(
        _sc_body,
        out_type=jax.ShapeDtypeStruct((NC, N, H), jnp.float32),
        mesh=mesh,
        scratch_types=[
            pltpu.VMEM((CHUNK,), jnp.int32),
            pltpu.VMEM((CHUNK,), jnp.int32),
            pltpu.VMEM((CHUNK,), jnp.int32),
            pltpu.VMEM((CHUNK,), jnp.float32),
            pltpu.VMEM((CHUNK,), jnp.int32),
            pltpu.VMEM((CHUNK, H), jnp.float32),
            pltpu.VMEM_SHARED((N, H), jnp.float32),
        ],
        compiler_params=pltpu.CompilerParams(needs_layout_passes=False),
    )
    return fn(table, src, dst, etypes, nrm)


def _tcA_body(nf_ref, emb_ref, wsm_ref, bsm_ref, co_ref, ba_ref, out_ref,
              x_scr):
    r = pl.program_id(1)

    @pl.when(r == 0)
    def _():
        nf = nf_ref[...]
        x = jnp.zeros((BLK, H), jnp.float32)
        for m in range(ML):
            oh = (nf[:, m][:, None] ==
                  lax.broadcasted_iota(jnp.int32, (1, VOCAB), 1))
            oh = oh.astype(jnp.float32)
            pm = jnp.dot(emb_ref[...], wsm_ref[pl.ds(m * EDIM, EDIM), :],
                         precision=_HIGH)
            x = x + jnp.dot(oh, pm, precision=_HIGH)
        x_scr[...] = x + bsm_ref[...][None, :]

    w = jnp.einsum("b,bio->io", co_ref[r], ba_ref[...], precision=_HIGH)
    out_ref[...] = jnp.dot(x_scr[...], w, precision=_HIGH)


def _tc_project1(node_feats, emb, W_sm, b_sm, coeff1, bases1):
    return pl.pallas_call(
        _tcA_body,
        grid=(NBLK, R),
        in_specs=[
            pl.BlockSpec((BLK, ML), lambda i, r: (i, 0)),
            pl.BlockSpec((VOCAB, EDIM), lambda i, r: (0, 0)),
            pl.BlockSpec((EDIM * ML, H), lambda i, r: (0, 0)),
            pl.BlockSpec((H,), lambda i, r: (0,)),
            pl.BlockSpec((R, NB), lambda i, r: (0, 0)),
            pl.BlockSpec((NB, H, H), lambda i, r: (0, 0, 0)),
        ],
        out_specs=pl.BlockSpec((BLK, H), lambda i, r: (r * NBLK + i, 0)),
        out_shape=jax.ShapeDtypeStruct((R * N, H), jnp.float32),
        scratch_shapes=[pltpu.VMEM((BLK, H), jnp.float32)],
    )(node_feats, emb, W_sm, b_sm, coeff1, bases1)


def _tcB_body(p_ref, b1_ref, co_ref, ba_ref, out_ref, h_scr):
    r = pl.program_id(1)

    @pl.when(r == 0)
    def _():
        h = p_ref[0] + p_ref[1] + b1_ref[...][None, :]
        h_scr[...] = jnp.maximum(h, 0.0)

    w = jnp.einsum("b,bio->io", co_ref[r], ba_ref[...], precision=_HIGH)
    out_ref[...] = jnp.dot(h_scr[...], w, precision=_HIGH)


def _tc_project2(p1, bias1, coeff2, bases2):
    return pl.pallas_call(
        _tcB_body,
        grid=(NBLK, R),
        in_specs=[
            pl.BlockSpec((NC, BLK, H), lambda i, r: (0, i, 0)),
            pl.BlockSpec((H,), lambda i, r: (0,)),
            pl.BlockSpec((R, NB), lambda i, r: (0, 0)),
            pl.BlockSpec((NB, H, OUT), lambda i, r: (0, 0, 0)),
        ],
        out_specs=pl.BlockSpec((BLK, OUT), lambda i, r: (r * NBLK + i, 0)),
        out_shape=jax.ShapeDtypeStruct((R * N, OUT), jnp.float32),
        scratch_shapes=[pltpu.VMEM((BLK, H), jnp.float32)],
    )(p1, bias1, coeff2, bases2)


def _tcC_body(p_ref, b2_ref, out_ref):
    out_ref[...] = p_ref[0] + p_ref[1] + b2_ref[...][None, :]


def _tc_merge(p2, bias2):
    return pl.pallas_call(
        _tcC_body,
        grid=(NBLK,),
        in_specs=[
            pl.BlockSpec((NC, BLK, OUT), lambda i: (0, i, 0)),
            pl.BlockSpec((OUT,), lambda i: (0,)),
        ],
        out_specs=pl.BlockSpec((BLK, OUT), lambda i: (i, 0)),
        out_shape=jax.ShapeDtypeStruct((N, OUT), jnp.float32),
    )(p2, bias2)


def kernel(emb, W_sm, b_sm, coeff1, bases1, bias1, coeff2, bases2, bias2,
           norm, node_feats, edge_index, etypes):
    src = edge_index[0]
    dst = edge_index[1]
    nrm = norm.reshape(E)

    xw1 = _tc_project1(node_feats, emb, W_sm, b_sm, coeff1, bases1)
    p1 = _sc_aggregate(xw1, src, dst, etypes, nrm)
    xw2 = _tc_project2(p1, bias1, coeff2, bases2)
    p2 = _sc_aggregate(xw2, src, dst, etypes, nrm)
    return _tc_merge(p2, bias2)


# speedup vs baseline: 9.6068x; 1.0777x over previous
import functools

import jax
import jax.numpy as jnp
from jax import lax
from jax.experimental import pallas as pl
from jax.experimental.pallas import tpu as pltpu
from jax.experimental.pallas import tpu_sc as plsc

N = 10000
E = 320000
R = 8
NB = 8
H = 128
EDIM = 16
ML = 8
VOCAB = 4
OUT = 128

NC = 2
NS = 16
NW = NC * NS
EPW = E // NW
CHUNK = 80
NCHUNK = EPW // CHUNK
ZROWS = 624
ZREM = N - NS * ZROWS

BLK = 1000
NBLK = N // BLK

_HIGH = jax.lax.Precision.HIGHEST


def _sc_body(table, src_h, dst_h, et_h, nrm_h, out_h,
             src_v, dst_v, et_v, nrm_v, gidx_v, rows_v, acc):
    c = lax.axis_index("c")
    s = lax.axis_index("s")
    wid = s * NC + c
    base = wid * EPW

    zeros16 = jnp.zeros((16,), jnp.float32)

    @pl.loop(0, CHUNK)
    def _zero_rows(i):
        for f in range(H // 16):
            rows_v[i, pl.ds(f * 16, 16)] = zeros16

    zbase = s * ZROWS
    nfull = ZROWS // CHUNK
    tail = ZROWS - nfull * CHUNK

    @pl.loop(0, nfull)
    def _zero_acc(j):
        pltpu.sync_copy(rows_v, acc.at[pl.ds(zbase + j * CHUNK, CHUNK)])

    pltpu.sync_copy(rows_v.at[pl.ds(0, tail)],
                    acc.at[pl.ds(zbase + nfull * CHUNK, tail)])

    @pl.when(s == NS - 1)
    def _zero_rem():
        pltpu.sync_copy(rows_v.at[pl.ds(0, ZREM)],
                        acc.at[pl.ds(NS * ZROWS, ZREM)])

    plsc.subcore_barrier()

    @pl.loop(0, NCHUNK)
    def _chunk(i):
        o = base + i * CHUNK
        pltpu.sync_copy(src_h.at[pl.ds(o, CHUNK)], src_v)
        pltpu.sync_copy(et_h.at[pl.ds(o, CHUNK)], et_v)
        pltpu.sync_copy(dst_h.at[pl.ds(o, CHUNK)], dst_v)
        pltpu.sync_copy(nrm_h.at[pl.ds(o, CHUNK)], nrm_v)

        for g in range(CHUNK // 16):
            sl = pl.ds(g * 16, 16)
            gidx_v[sl] = et_v[sl] * N + src_v[sl]

        pltpu.sync_copy(table.at[gidx_v], rows_v)

        @plsc.parallel_loop(0, CHUNK // 16, unroll=1)
        def _scale(g):
            g16 = nrm_v[pl.ds(g * 16, 16)]
            for j in range(16):
                nb = jnp.take_along_axis(g16, jnp.full((16,), j, jnp.int32),
                                         axis=0, mode="promise_in_bounds")
                e = g * 16 + j
                for f in range(H // 16):
                    sl = pl.ds(f * 16, 16)
                    rows_v[e, sl] = rows_v[e, sl] * nb

        pltpu.sync_copy(rows_v, acc.at[dst_v], add=True)

    plsc.subcore_barrier()

    pltpu.sync_copy(acc.at[pl.ds(zbase, ZROWS)],
                    out_h.at[c, pl.ds(zbase, ZROWS)])

    @pl.when(s == NS - 1)
    def _wb_rem():
        pltpu.sync_copy(acc.at[pl.ds(NS * ZROWS, ZREM)],
                        out_h.at[c, pl.ds(NS * ZROWS, ZREM)])


def _sc_aggregate(table, src, dst, etypes, nrm):
    mesh = plsc.VectorSubcoreMesh(core_axis_name="c", subcore_axis_name="s",
                                  num_cores=NC, num_subcores=NS)
    fn = pl.kernel(
        _sc_body,
        out_type=jax.ShapeDtypeStruct((NC, N, H), jnp.float32),
        mesh=mesh,
        scratch_types=[
            pltpu.VMEM((CHUNK,), jnp.int32),
            pltpu.VMEM((CHUNK,), jnp.int32),
            pltpu.VMEM((CHUNK,), jnp.int32),
            pltpu.VMEM((CHUNK,), jnp.float32),
            pltpu.VMEM((CHUNK,), jnp.int32),
            pltpu.VMEM((CHUNK, H), jnp.float32),
            pltpu.VMEM_SHARED((N, H), jnp.float32),
        ],
        compiler_params=pltpu.CompilerParams(needs_layout_passes=False),
    )
    return fn(table, src, dst, etypes, nrm)


def _tcA_body(nf_ref, emb_ref, wsm_ref, bsm_ref, co_ref, ba_ref, out_ref,
              x_scr):
    r = pl.program_id(1)

    @pl.when(r == 0)
    def _():
        nf = nf_ref[...]
        x = jnp.zeros((BLK, H), jnp.float32)
        for m in range(ML):
            oh = (nf[:, m][:, None] ==
                  lax.broadcasted_iota(jnp.int32, (1, VOCAB), 1))
            oh = oh.astype(jnp.float32)
            pm = jnp.dot(emb_ref[...], wsm_ref[pl.ds(m * EDIM, EDIM), :],
                         precision=_HIGH)
            x = x + jnp.dot(oh, pm, precision=_HIGH)
        x_scr[...] = x + bsm_ref[...][None, :]

    w = jnp.einsum("b,bio->io", co_ref[r], ba_ref[...], precision=_HIGH)
    out_ref[...] = jnp.dot(x_scr[...], w, precision=_HIGH)


def _tc_project1(node_feats, emb, W_sm, b_sm, coeff1, bases1):
    return pl.pallas_call(
        _tcA_body,
        grid=(NBLK, R),
        in_specs=[
            pl.BlockSpec((BLK, ML), lambda i, r: (i, 0)),
            pl.BlockSpec((VOCAB, EDIM), lambda i, r: (0, 0)),
            pl.BlockSpec((EDIM * ML, H), lambda i, r: (0, 0)),
            pl.BlockSpec((H,), lambda i, r: (0,)),
            pl.BlockSpec((R, NB), lambda i, r: (0, 0)),
            pl.BlockSpec((NB, H, H), lambda i, r: (0, 0, 0)),
        ],
        out_specs=pl.BlockSpec((BLK, H), lambda i, r: (r * NBLK + i, 0)),
        out_shape=jax.ShapeDtypeStruct((R * N, H), jnp.float32),
        scratch_shapes=[pltpu.VMEM((BLK, H), jnp.float32)],
    )(node_feats, emb, W_sm, b_sm, coeff1, bases1)


def _tcB_body(p_ref, b1_ref, co_ref, ba_ref, out_ref, h_scr):
    r = pl.program_id(1)

    @pl.when(r == 0)
    def _():
        h = p_ref[0] + p_ref[1] + b1_ref[...][None, :]
        h_scr[...] = jnp.maximum(h, 0.0)

    w = jnp.einsum("b,bio->io", co_ref[r], ba_ref[...], precision=_HIGH)
    out_ref[...] = jnp.dot(h_scr[...], w, precision=_HIGH)


def _tc_project2(p1, bias1, coeff2, bases2):
    return pl.pallas_call(
        _tcB_body,
        grid=(NBLK, R),
        in_specs=[
            pl.BlockSpec((NC, BLK, H), lambda i, r: (0, i, 0)),
            pl.BlockSpec((H,), lambda i, r: (0,)),
            pl.BlockSpec((R, NB), lambda i, r: (0, 0)),
            pl.BlockSpec((NB, H, OUT), lambda i, r: (0, 0, 0)),
        ],
        out_specs=pl.BlockSpec((BLK, OUT), lambda i, r: (r * NBLK + i, 0)),
        out_shape=jax.ShapeDtypeStruct((R * N, OUT), jnp.float32),
        scratch_shapes=[pltpu.VMEM((BLK, H), jnp.float32)],
    )(p1, bias1, coeff2, bases2)


def _tcC_body(p_ref, b2_ref, out_ref):
    out_ref[...] = p_ref[0] + p_ref[1] + b2_ref[...][None, :]


def _tc_merge(p2, bias2):
    return pl.pallas_call(
        _tcC_body,
        grid=(NBLK,),
        in_specs=[
            pl.BlockSpec((NC, BLK, OUT), lambda i: (0, i, 0)),
            pl.BlockSpec((OUT,), lambda i: (0,)),
        ],
        out_specs=pl.BlockSpec((BLK, OUT), lambda i: (i, 0)),
        out_shape=jax.ShapeDtypeStruct((N, OUT), jnp.float32),
    )(p2, bias2)


def kernel(emb, W_sm, b_sm, coeff1, bases1, bias1, coeff2, bases2, bias2,
           norm, node_feats, edge_index, etypes):
    src = edge_index[0]
    dst = edge_index[1]
    nrm = norm.reshape(E)

    xw1 = _tc_project1(node_feats, emb, W_sm, b_sm, coeff1, bases1)
    p1 = _sc_aggregate(xw1, src, dst, etypes, nrm)
    xw2 = _tc_project2(p1, bias1, coeff2, bases2)
    p2 = _sc_aggregate(xw2, src, dst, etypes, nrm)
    return _tc_merge(p2, bias2)


# speedup vs baseline: 16.2157x; 1.6879x over previous
import functools

import jax
import jax.numpy as jnp
from jax import lax
from jax.experimental import pallas as pl
from jax.experimental.pallas import tpu as pltpu
from jax.experimental.pallas import tpu_sc as plsc

N = 10000
E = 320000
R = 8
NB = 8
H = 128
EDIM = 16
ML = 8
VOCAB = 4
OUT = 128

NC = 2
NS = 16
NW = NC * NS
EPW = E // NW
CHUNK = 80
NCHUNK = EPW // CHUNK
ZROWS = 624
ZREM = N - NS * ZROWS

BLK = 1000
NBLK = N // BLK

_HIGH = jax.lax.Precision.HIGHEST


def _sc_body(table, src_h, dst_h, et_h, nrm_h, out_h,
             src_v, dst_v, et_v, nrm_v, gidx_v, rows_v, acc,
             isem0, isem1, gsem0, gsem1):
    c = lax.axis_index("c")
    s = lax.axis_index("s")
    wid = s * NC + c
    base = wid * EPW
    isems = (isem0, isem1)
    gsems = (gsem0, gsem1)

    zeros16 = jnp.zeros((16,), jnp.float32)

    @pl.loop(0, CHUNK)
    def _zero_rows(i):
        for f in range(H // 16):
            rows_v[0, i, pl.ds(f * 16, 16)] = zeros16

    zbase = s * ZROWS
    nfull = ZROWS // CHUNK
    tail = ZROWS - nfull * CHUNK

    @pl.loop(0, nfull)
    def _zero_acc(j):
        pltpu.sync_copy(rows_v.at[0], acc.at[pl.ds(zbase + j * CHUNK, CHUNK)])

    pltpu.sync_copy(rows_v.at[0, pl.ds(0, tail)],
                    acc.at[pl.ds(zbase + nfull * CHUNK, tail)])

    @pl.when(s == NS - 1)
    def _zero_rem():
        pltpu.sync_copy(rows_v.at[0, pl.ds(0, ZREM)],
                        acc.at[pl.ds(NS * ZROWS, ZREM)])

    plsc.subcore_barrier()


    def issue_idx(ch, b):
        o = base + ch * CHUNK
        pltpu.async_copy(src_h.at[pl.ds(o, CHUNK)], src_v.at[b], isems[b])
        pltpu.async_copy(et_h.at[pl.ds(o, CHUNK)], et_v.at[b], isems[b])
        pltpu.async_copy(dst_h.at[pl.ds(o, CHUNK)], dst_v.at[b], isems[b])
        pltpu.async_copy(nrm_h.at[pl.ds(o, CHUNK)], nrm_v.at[b], isems[b])

    def drain_idx(b):
        o = base
        for ref in (src_v, et_v, dst_v, nrm_v):
            pltpu.make_async_copy(src_h.at[pl.ds(o, CHUNK)], ref.at[b],
                                  isems[b]).wait()

    def compute_gidx(b):
        for g in range(CHUNK // 16):
            sl = pl.ds(g * 16, 16)
            gidx_v[b, sl] = et_v[b, sl] * N + src_v[b, sl]

    def issue_gather(b):
        pltpu.async_copy(table.at[gidx_v.at[b]], rows_v.at[b], gsems[b])

    def wait_gather(b):
        pltpu.make_async_copy(table.at[gidx_v.at[b]], rows_v.at[b],
                              gsems[b]).wait()

    def scale_scatter(b):
        @plsc.parallel_loop(0, CHUNK // 16, unroll=1)
        def _scale(g):
            g16 = nrm_v[b, pl.ds(g * 16, 16)]
            for j in range(16):
                nb = jnp.take_along_axis(g16, jnp.full((16,), j, jnp.int32),
                                         axis=0, mode="promise_in_bounds")
                e = g * 16 + j
                for f in range(H // 16):
                    sl = pl.ds(f * 16, 16)
                    rows_v[b, e, sl] = rows_v[b, e, sl] * nb

        pltpu.sync_copy(rows_v.at[b], acc.at[dst_v.at[b]], add=True)

    def half(c0, b):
        @pl.when(c0 + 1 < NCHUNK)
        def _():
            drain_idx(1 - b)
            compute_gidx(1 - b)
            issue_gather(1 - b)
        wait_gather(b)
        scale_scatter(b)
        @pl.when(c0 + 2 < NCHUNK)
        def _():
            issue_idx(c0 + 2, b)

    issue_idx(0, 0)
    drain_idx(0)
    compute_gidx(0)
    issue_gather(0)
    issue_idx(1, 1)

    @pl.loop(0, NCHUNK // 2)
    def _pair(i2):
        half(2 * i2, 0)
        half(2 * i2 + 1, 1)

    half(NCHUNK - 1, 0)

    plsc.subcore_barrier()

    pltpu.sync_copy(acc.at[pl.ds(zbase, ZROWS)],
                    out_h.at[c, pl.ds(zbase, ZROWS)])

    @pl.when(s == NS - 1)
    def _wb_rem():
        pltpu.sync_copy(acc.at[pl.ds(NS * ZROWS, ZREM)],
                        out_h.at[c, pl.ds(NS * ZROWS, ZREM)])


def _sc_aggregate(table, src, dst, etypes, nrm):
    mesh = plsc.VectorSubcoreMesh(core_axis_name="c", subcore_axis_name="s",
                                  num_cores=NC, num_subcores=NS)
    fn = pl.kernel(
        _sc_body,
        out_type=jax.ShapeDtypeStruct((NC, N, H), jnp.float32),
        mesh=mesh,
        scratch_types=[
            pltpu.VMEM((2, CHUNK), jnp.int32),
            pltpu.VMEM((2, CHUNK), jnp.int32),
            pltpu.VMEM((2, CHUNK), jnp.int32),
            pltpu.VMEM((2, CHUNK), jnp.float32),
            pltpu.VMEM((2, CHUNK), jnp.int32),
            pltpu.VMEM((2, CHUNK, H), jnp.float32),
            pltpu.VMEM_SHARED((N, H), jnp.float32),
            pltpu.SemaphoreType.DMA,
            pltpu.SemaphoreType.DMA,
            pltpu.SemaphoreType.DMA,
            pltpu.SemaphoreType.DMA,
        ],
        compiler_params=pltpu.CompilerParams(needs_layout_passes=False),
    )
    return fn(table, src, dst, etypes, nrm)


def _tcA_body(nf_ref, emb_ref, wsm_ref, bsm_ref, co_ref, ba_ref, out_ref,
              x_scr):
    r = pl.program_id(1)

    @pl.when(r == 0)
    def _():
        nf = nf_ref[...]
        x = jnp.zeros((BLK, H), jnp.float32)
        for m in range(ML):
            oh = (nf[:, m][:, None] ==
                  lax.broadcasted_iota(jnp.int32, (1, VOCAB), 1))
            oh = oh.astype(jnp.float32)
            pm = jnp.dot(emb_ref[...], wsm_ref[pl.ds(m * EDIM, EDIM), :],
                         precision=_HIGH)
            x = x + jnp.dot(oh, pm, precision=_HIGH)
        x_scr[...] = x + bsm_ref[...][None, :]

    w = jnp.einsum("b,bio->io", co_ref[r], ba_ref[...], precision=_HIGH)
    out_ref[...] = jnp.dot(x_scr[...], w, precision=_HIGH)


def _tc_project1(node_feats, emb, W_sm, b_sm, coeff1, bases1):
    return pl.pallas_call(
        _tcA_body,
        grid=(NBLK, R),
        in_specs=[
            pl.BlockSpec((BLK, ML), lambda i, r: (i, 0)),
            pl.BlockSpec((VOCAB, EDIM), lambda i, r: (0, 0)),
            pl.BlockSpec((EDIM * ML, H), lambda i, r: (0, 0)),
            pl.BlockSpec((H,), lambda i, r: (0,)),
            pl.BlockSpec((R, NB), lambda i, r: (0, 0)),
            pl.BlockSpec((NB, H, H), lambda i, r: (0, 0, 0)),
        ],
        out_specs=pl.BlockSpec((BLK, H), lambda i, r: (r * NBLK + i, 0)),
        out_shape=jax.ShapeDtypeStruct((R * N, H), jnp.float32),
        scratch_shapes=[pltpu.VMEM((BLK, H), jnp.float32)],
    )(node_feats, emb, W_sm, b_sm, coeff1, bases1)


def _tcB_body(p_ref, b1_ref, co_ref, ba_ref, out_ref, h_scr):
    r = pl.program_id(1)

    @pl.when(r == 0)
    def _():
        h = p_ref[0] + p_ref[1] + b1_ref[...][None, :]
        h_scr[...] = jnp.maximum(h, 0.0)

    w = jnp.einsum("b,bio->io", co_ref[r], ba_ref[...], precision=_HIGH)
    out_ref[...] = jnp.dot(h_scr[...], w, precision=_HIGH)


def _tc_project2(p1, bias1, coeff2, bases2):
    return pl.pallas_call(
        _tcB_body,
        grid=(NBLK, R),
        in_specs=[
            pl.BlockSpec((NC, BLK, H), lambda i, r: (0, i, 0)),
            pl.BlockSpec((H,), lambda i, r: (0,)),
            pl.BlockSpec((R, NB), lambda i, r: (0, 0)),
            pl.BlockSpec((NB, H, OUT), lambda i, r: (0, 0, 0)),
        ],
        out_specs=pl.BlockSpec((BLK, OUT), lambda i, r: (r * NBLK + i, 0)),
        out_shape=jax.ShapeDtypeStruct((R * N, OUT), jnp.float32),
        scratch_shapes=[pltpu.VMEM((BLK, H), jnp.float32)],
    )(p1, bias1, coeff2, bases2)


def _tcC_body(p_ref, b2_ref, out_ref):
    out_ref[...] = p_ref[0] + p_ref[1] + b2_ref[...][None, :]


def _tc_merge(p2, bias2):
    return pl.pallas_call(
        _tcC_body,
        grid=(NBLK,),
        in_specs=[
            pl.BlockSpec((NC, BLK, OUT), lambda i: (0, i, 0)),
            pl.BlockSpec((OUT,), lambda i: (0,)),
        ],
        out_specs=pl.BlockSpec((BLK, OUT), lambda i: (i, 0)),
        out_shape=jax.ShapeDtypeStruct((N, OUT), jnp.float32),
    )(p2, bias2)


def kernel(emb, W_sm, b_sm, coeff1, bases1, bias1, coeff2, bases2, bias2,
           norm, node_feats, edge_index, etypes):
    src = edge_index[0]
    dst = edge_index[1]
    nrm = norm.reshape(E)

    xw1 = _tc_project1(node_feats, emb, W_sm, b_sm, coeff1, bases1)
    p1 = _sc_aggregate(xw1, src, dst, etypes, nrm)
    xw2 = _tc_project2(p1, bias1, coeff2, bases2)
    p2 = _sc_aggregate(xw2, src, dst, etypes, nrm)
    return _tc_merge(p2, bias2)


# speedup vs baseline: 17.3145x; 1.0678x over previous
import functools

import jax
import jax.numpy as jnp
from jax import lax
from jax.experimental import pallas as pl
from jax.experimental.pallas import tpu as pltpu
from jax.experimental.pallas import tpu_sc as plsc

N = 10000
E = 320000
R = 8
NB = 8
H = 128
EDIM = 16
ML = 8
VOCAB = 4
OUT = 128

NC = 2
NS = 16
NW = NC * NS
EPW = E // NW
CHUNK = 80
NCHUNK = EPW // CHUNK
ZROWS = 624
ZREM = N - NS * ZROWS

BLK = 1000
NBLK = N // BLK

_HIGH = jax.lax.Precision.HIGHEST


def _sc_body(table, src_h, dst_h, et_h, nrm_h, out_h,
             src_v, dst_v, et_v, nrm_v, gidx_v, rows_v, acc,
             isem0, isem1, gsem0, gsem1):
    c = lax.axis_index("c")
    s = lax.axis_index("s")
    wid = s * NC + c
    base = wid * EPW
    isems = (isem0, isem1)
    gsems = (gsem0, gsem1)

    zeros16 = jnp.zeros((16,), jnp.float32)

    @pl.loop(0, CHUNK)
    def _zero_rows(i):
        for f in range(H // 16):
            rows_v[0, i, pl.ds(f * 16, 16)] = zeros16

    zbase = s * ZROWS
    nfull = ZROWS // CHUNK
    tail = ZROWS - nfull * CHUNK

    @pl.loop(0, nfull)
    def _zero_acc(j):
        pltpu.sync_copy(rows_v.at[0], acc.at[pl.ds(zbase + j * CHUNK, CHUNK)])

    pltpu.sync_copy(rows_v.at[0, pl.ds(0, tail)],
                    acc.at[pl.ds(zbase + nfull * CHUNK, tail)])

    @pl.when(s == NS - 1)
    def _zero_rem():
        pltpu.sync_copy(rows_v.at[0, pl.ds(0, ZREM)],
                        acc.at[pl.ds(NS * ZROWS, ZREM)])

    plsc.subcore_barrier()


    def issue_idx(ch, b):
        o = base + ch * CHUNK
        pltpu.async_copy(src_h.at[pl.ds(o, CHUNK)], src_v.at[b], isems[b])
        pltpu.async_copy(et_h.at[pl.ds(o, CHUNK)], et_v.at[b], isems[b])
        pltpu.async_copy(dst_h.at[pl.ds(o, CHUNK)], dst_v.at[b], isems[b])
        pltpu.async_copy(nrm_h.at[pl.ds(o, CHUNK)], nrm_v.at[b], isems[b])

    def drain_idx(b):
        o = base
        for ref in (src_v, et_v, dst_v, nrm_v):
            pltpu.make_async_copy(src_h.at[pl.ds(o, CHUNK)], ref.at[b],
                                  isems[b]).wait()

    def compute_gidx(b):
        for g in range(CHUNK // 16):
            sl = pl.ds(g * 16, 16)
            gidx_v[b, sl] = et_v[b, sl] * N + src_v[b, sl]

    def issue_gather(b):
        pltpu.async_copy(table.at[gidx_v.at[b]], rows_v.at[b], gsems[b])

    def wait_gather(b):
        pltpu.make_async_copy(table.at[gidx_v.at[b]], rows_v.at[b],
                              gsems[b]).wait()

    def scale_scatter(b):
        @plsc.parallel_loop(0, CHUNK // 16, unroll=1)
        def _scale(g):
            g16 = nrm_v[b, pl.ds(g * 16, 16)]
            for j in range(16):
                nb = jnp.take_along_axis(g16, jnp.full((16,), j, jnp.int32),
                                         axis=0, mode="promise_in_bounds")
                e = g * 16 + j
                for f in range(H // 16):
                    sl = pl.ds(f * 16, 16)
                    rows_v[b, e, sl] = rows_v[b, e, sl] * nb

        pltpu.sync_copy(rows_v.at[b], acc.at[dst_v.at[b]], add=True)

    def half(c0, b):
        @pl.when(c0 + 1 < NCHUNK)
        def _():
            drain_idx(1 - b)
            compute_gidx(1 - b)
            issue_gather(1 - b)
        wait_gather(b)
        scale_scatter(b)
        @pl.when(c0 + 2 < NCHUNK)
        def _():
            issue_idx(c0 + 2, b)

    issue_idx(0, 0)
    drain_idx(0)
    compute_gidx(0)
    issue_gather(0)
    issue_idx(1, 1)

    @pl.loop(0, NCHUNK // 2)
    def _pair(i2):
        half(2 * i2, 0)
        half(2 * i2 + 1, 1)

    half(NCHUNK - 1, 0)

    plsc.subcore_barrier()

    pltpu.sync_copy(acc.at[pl.ds(zbase, ZROWS)],
                    out_h.at[c, pl.ds(zbase, ZROWS)])

    @pl.when(s == NS - 1)
    def _wb_rem():
        pltpu.sync_copy(acc.at[pl.ds(NS * ZROWS, ZREM)],
                        out_h.at[c, pl.ds(NS * ZROWS, ZREM)])


def _sc_aggregate(table, src, dst, etypes, nrm):
    mesh = plsc.VectorSubcoreMesh(core_axis_name="c", subcore_axis_name="s",
                                  num_cores=NC, num_subcores=NS)
    fn = pl.kernel(
        _sc_body,
        out_type=jax.ShapeDtypeStruct((NC, N, H), jnp.float32),
        mesh=mesh,
        scratch_types=[
            pltpu.VMEM((2, CHUNK), jnp.int32),
            pltpu.VMEM((2, CHUNK), jnp.int32),
            pltpu.VMEM((2, CHUNK), jnp.int32),
            pltpu.VMEM((2, CHUNK), jnp.float32),
            pltpu.VMEM((2, CHUNK), jnp.int32),
            pltpu.VMEM((2, CHUNK, H), jnp.float32),
            pltpu.VMEM_SHARED((N, H), jnp.float32),
            pltpu.SemaphoreType.DMA,
            pltpu.SemaphoreType.DMA,
            pltpu.SemaphoreType.DMA,
            pltpu.SemaphoreType.DMA,
        ],
        compiler_params=pltpu.CompilerParams(needs_layout_passes=False),
    )
    return fn(table, src, dst, etypes, nrm)


def _tcA_body(nf_ref, emb_ref, wsm_ref, bsm_ref, co_ref, ba_ref, out_ref,
              x_scr):
    r = pl.program_id(1)

    @pl.when(r == 0)
    def _():
        nf = nf_ref[...]
        col_m = lax.broadcasted_iota(jnp.int32, (BLK, ML * VOCAB), 1) // VOCAB
        col_v = lax.broadcasted_iota(jnp.int32, (BLK, ML * VOCAB), 1) % VOCAB
        nf_g = jnp.take_along_axis(nf, col_m, axis=1)
        oh = (nf_g == col_v).astype(jnp.float32)
        wsm3 = wsm_ref[...].reshape(ML, EDIM, H)
        m2 = jnp.einsum("vd,mdh->mvh", emb_ref[...], wsm3,
                        precision=_HIGH).reshape(ML * VOCAB, H)
        x_scr[...] = (jnp.dot(oh, m2, precision=_HIGH)
                      + bsm_ref[...][None, :])

    w = jnp.einsum("b,bio->io", co_ref[r], ba_ref[...], precision=_HIGH)
    out_ref[...] = jnp.dot(x_scr[...], w, precision=_HIGH)


def _tc_project1(node_feats, emb, W_sm, b_sm, coeff1, bases1):
    return pl.pallas_call(
        _tcA_body,
        grid=(NBLK, R),
        in_specs=[
            pl.BlockSpec((BLK, ML), lambda i, r: (i, 0)),
            pl.BlockSpec((VOCAB, EDIM), lambda i, r: (0, 0)),
            pl.BlockSpec((EDIM * ML, H), lambda i, r: (0, 0)),
            pl.BlockSpec((H,), lambda i, r: (0,)),
            pl.BlockSpec((R, NB), lambda i, r: (0, 0)),
            pl.BlockSpec((NB, H, H), lambda i, r: (0, 0, 0)),
        ],
        out_specs=pl.BlockSpec((BLK, H), lambda i, r: (r * NBLK + i, 0)),
        out_shape=jax.ShapeDtypeStruct((R * N, H), jnp.float32),
        scratch_shapes=[pltpu.VMEM((BLK, H), jnp.float32)],
    )(node_feats, emb, W_sm, b_sm, coeff1, bases1)


def _tcB_body(p_ref, b1_ref, co_ref, ba_ref, out_ref, h_scr):
    r = pl.program_id(1)

    @pl.when(r == 0)
    def _():
        h = p_ref[0] + p_ref[1] + b1_ref[...][None, :]
        h_scr[...] = jnp.maximum(h, 0.0)

    w = jnp.einsum("b,bio->io", co_ref[r], ba_ref[...], precision=_HIGH)
    out_ref[...] = jnp.dot(h_scr[...], w, precision=_HIGH)


def _tc_project2(p1, bias1, coeff2, bases2):
    return pl.pallas_call(
        _tcB_body,
        grid=(NBLK, R),
        in_specs=[
            pl.BlockSpec((NC, BLK, H), lambda i, r: (0, i, 0)),
            pl.BlockSpec((H,), lambda i, r: (0,)),
            pl.BlockSpec((R, NB), lambda i, r: (0, 0)),
            pl.BlockSpec((NB, H, OUT), lambda i, r: (0, 0, 0)),
        ],
        out_specs=pl.BlockSpec((BLK, OUT), lambda i, r: (r * NBLK + i, 0)),
        out_shape=jax.ShapeDtypeStruct((R * N, OUT), jnp.float32),
        scratch_shapes=[pltpu.VMEM((BLK, H), jnp.float32)],
    )(p1, bias1, coeff2, bases2)


def _tcC_body(p_ref, b2_ref, out_ref):
    out_ref[...] = p_ref[0] + p_ref[1] + b2_ref[...][None, :]


def _tc_merge(p2, bias2):
    return pl.pallas_call(
        _tcC_body,
        grid=(NBLK,),
        in_specs=[
            pl.BlockSpec((NC, BLK, OUT), lambda i: (0, i, 0)),
            pl.BlockSpec((OUT,), lambda i: (0,)),
        ],
        out_specs=pl.BlockSpec((BLK, OUT), lambda i: (i, 0)),
        out_shape=jax.ShapeDtypeStruct((N, OUT), jnp.float32),
    )(p2, bias2)


def kernel(emb, W_sm, b_sm, coeff1, bases1, bias1, coeff2, bases2, bias2,
           norm, node_feats, edge_index, etypes):
    src = edge_index[0]
    dst = edge_index[1]
    nrm = norm.reshape(E)

    xw1 = _tc_project1(node_feats, emb, W_sm, b_sm, coeff1, bases1)
    p1 = _sc_aggregate(xw1, src, dst, etypes, nrm)
    xw2 = _tc_project2(p1, bias1, coeff2, bases2)
    p2 = _sc_aggregate(xw2, src, dst, etypes, nrm)
    return _tc_merge(p2, bias2)


# speedup vs baseline: 21.0172x; 1.2138x over previous
import functools

import jax
import jax.numpy as jnp
from jax import lax
from jax.experimental import pallas as pl
from jax.experimental.pallas import tpu as pltpu
from jax.experimental.pallas import tpu_sc as plsc

N = 10000
E = 320000
R = 8
NB = 8
H = 128
EDIM = 16
ML = 8
VOCAB = 4
OUT = 128

NC = 2
NS = 16
NW = NC * NS
EPW = E // NW
CHUNK = 80
NCHUNK = EPW // CHUNK
ZROWS = 624
ZREM = N - NS * ZROWS

BLK = 1000
NBLK = N // BLK

_HIGH = jax.lax.Precision.HIGHEST


def _sc_body(table, src_h, dst_h, et_h, nrm_h, out_h,
             src_v, dst_v, et_v, nrm_v, gidx_v, rows_v, acc,
             isem0, isem1, gsem0, gsem1):
    c = lax.axis_index("c")
    s = lax.axis_index("s")
    wid = s * NC + c
    base = wid * EPW
    isems = (isem0, isem1)
    gsems = (gsem0, gsem1)

    zeros16 = jnp.zeros((16,), jnp.float32)

    @pl.loop(0, CHUNK)
    def _zero_rows(i):
        for f in range(H // 16):
            rows_v[0, i, pl.ds(f * 16, 16)] = zeros16

    zbase = s * ZROWS
    nfull = ZROWS // CHUNK
    tail = ZROWS - nfull * CHUNK

    @pl.loop(0, nfull)
    def _zero_acc(j):
        pltpu.sync_copy(rows_v.at[0], acc.at[pl.ds(zbase + j * CHUNK, CHUNK)])

    pltpu.sync_copy(rows_v.at[0, pl.ds(0, tail)],
                    acc.at[pl.ds(zbase + nfull * CHUNK, tail)])

    @pl.when(s == NS - 1)
    def _zero_rem():
        pltpu.sync_copy(rows_v.at[0, pl.ds(0, ZREM)],
                        acc.at[pl.ds(NS * ZROWS, ZREM)])

    plsc.subcore_barrier()


    def issue_idx(ch, b):
        o = base + ch * CHUNK
        pltpu.async_copy(src_h.at[pl.ds(o, CHUNK)], src_v.at[b], isems[b])
        pltpu.async_copy(et_h.at[pl.ds(o, CHUNK)], et_v.at[b], isems[b])
        pltpu.async_copy(dst_h.at[pl.ds(o, CHUNK)], dst_v.at[b], isems[b])
        pltpu.async_copy(nrm_h.at[pl.ds(o, CHUNK)], nrm_v.at[b], isems[b])

    def drain_idx(b):
        o = base
        for ref in (src_v, et_v, dst_v, nrm_v):
            pltpu.make_async_copy(src_h.at[pl.ds(o, CHUNK)], ref.at[b],
                                  isems[b]).wait()

    def compute_gidx(b):
        for g in range(CHUNK // 16):
            sl = pl.ds(g * 16, 16)
            gidx_v[b, sl] = et_v[b, sl] * N + src_v[b, sl]

    def issue_gather(b):
        pltpu.async_copy(table.at[gidx_v.at[b]], rows_v.at[b], gsems[b])

    def wait_gather(b):
        pltpu.make_async_copy(table.at[gidx_v.at[b]], rows_v.at[b],
                              gsems[b]).wait()

    def scale_scatter(b):
        @plsc.parallel_loop(0, CHUNK // 16, unroll=1)
        def _scale(g):
            g16 = nrm_v[b, pl.ds(g * 16, 16)]
            for j in range(16):
                nb = jnp.take_along_axis(g16, jnp.full((16,), j, jnp.int32),
                                         axis=0, mode="promise_in_bounds")
                e = g * 16 + j
                for f in range(H // 16):
                    sl = pl.ds(f * 16, 16)
                    rows_v[b, e, sl] = rows_v[b, e, sl] * nb

        pltpu.sync_copy(rows_v.at[b], acc.at[dst_v.at[b]], add=True)

    def half(c0, b):
        @pl.when(c0 + 1 < NCHUNK)
        def _():
            drain_idx(1 - b)
            compute_gidx(1 - b)
            issue_gather(1 - b)
        wait_gather(b)
        scale_scatter(b)
        @pl.when(c0 + 2 < NCHUNK)
        def _():
            issue_idx(c0 + 2, b)

    issue_idx(0, 0)
    drain_idx(0)
    compute_gidx(0)
    issue_gather(0)
    issue_idx(1, 1)

    @pl.loop(0, NCHUNK // 2)
    def _pair(i2):
        half(2 * i2, 0)
        half(2 * i2 + 1, 1)

    half(NCHUNK - 1, 0)

    plsc.subcore_barrier()

    pltpu.sync_copy(acc.at[pl.ds(zbase, ZROWS)],
                    out_h.at[c, pl.ds(zbase, ZROWS)])

    @pl.when(s == NS - 1)
    def _wb_rem():
        pltpu.sync_copy(acc.at[pl.ds(NS * ZROWS, ZREM)],
                        out_h.at[c, pl.ds(NS * ZROWS, ZREM)])


def _sc_aggregate(table, src, dst, etypes, nrm):
    mesh = plsc.VectorSubcoreMesh(core_axis_name="c", subcore_axis_name="s",
                                  num_cores=NC, num_subcores=NS)
    fn = pl.kernel(
        _sc_body,
        out_type=jax.ShapeDtypeStruct((NC, N, H), jnp.float32),
        mesh=mesh,
        scratch_types=[
            pltpu.VMEM((2, CHUNK), jnp.int32),
            pltpu.VMEM((2, CHUNK), jnp.int32),
            pltpu.VMEM((2, CHUNK), jnp.int32),
            pltpu.VMEM((2, CHUNK), jnp.float32),
            pltpu.VMEM((2, CHUNK), jnp.int32),
            pltpu.VMEM((2, CHUNK, H), jnp.float32),
            pltpu.VMEM_SHARED((N, H), jnp.float32),
            pltpu.SemaphoreType.DMA,
            pltpu.SemaphoreType.DMA,
            pltpu.SemaphoreType.DMA,
            pltpu.SemaphoreType.DMA,
        ],
        compiler_params=pltpu.CompilerParams(needs_layout_passes=False),
    )
    return fn(table, src, dst, etypes, nrm)


def _tcA_body(nf_ref, emb_ref, wsm_ref, bsm_ref, co_ref, ba_ref, out_ref,
              x_scr, w_scr):
    i = pl.program_id(0)
    r = pl.program_id(1)

    @pl.when(i == 0)
    def _():
        w_scr[r] = jnp.einsum("b,bio->io", co_ref[r], ba_ref[...],
                              precision=_HIGH)

    @pl.when(r == 0)
    def _():
        nf = nf_ref[...]
        col_m = lax.broadcasted_iota(jnp.int32, (BLK, ML * VOCAB), 1) // VOCAB
        col_v = lax.broadcasted_iota(jnp.int32, (BLK, ML * VOCAB), 1) % VOCAB
        nf_g = jnp.take_along_axis(nf, col_m, axis=1)
        oh = (nf_g == col_v).astype(jnp.float32)
        wsm3 = wsm_ref[...].reshape(ML, EDIM, H)
        m2 = jnp.einsum("vd,mdh->mvh", emb_ref[...], wsm3,
                        precision=_HIGH).reshape(ML * VOCAB, H)
        x_scr[...] = (jnp.dot(oh, m2, precision=_HIGH)
                      + bsm_ref[...][None, :])

    out_ref[...] = jnp.dot(x_scr[...], w_scr[r], precision=_HIGH)


def _tc_project1(node_feats, emb, W_sm, b_sm, coeff1, bases1):
    return pl.pallas_call(
        _tcA_body,
        grid=(NBLK, R),
        in_specs=[
            pl.BlockSpec((BLK, ML), lambda i, r: (i, 0)),
            pl.BlockSpec((VOCAB, EDIM), lambda i, r: (0, 0)),
            pl.BlockSpec((EDIM * ML, H), lambda i, r: (0, 0)),
            pl.BlockSpec((H,), lambda i, r: (0,)),
            pl.BlockSpec((R, NB), lambda i, r: (0, 0)),
            pl.BlockSpec((NB, H, H), lambda i, r: (0, 0, 0)),
        ],
        out_specs=pl.BlockSpec((BLK, H), lambda i, r: (r * NBLK + i, 0)),
        out_shape=jax.ShapeDtypeStruct((R * N, H), jnp.float32),
        scratch_shapes=[pltpu.VMEM((BLK, H), jnp.float32),
                        pltpu.VMEM((R, H, H), jnp.float32)],
    )(node_feats, emb, W_sm, b_sm, coeff1, bases1)


def _tcB_body(p_ref, b1_ref, co_ref, ba_ref, out_ref, h_scr, w_scr):
    i = pl.program_id(0)
    r = pl.program_id(1)

    @pl.when(i == 0)
    def _():
        w_scr[r] = jnp.einsum("b,bio->io", co_ref[r], ba_ref[...],
                              precision=_HIGH)

    @pl.when(r == 0)
    def _():
        h = p_ref[0] + p_ref[1] + b1_ref[...][None, :]
        h_scr[...] = jnp.maximum(h, 0.0)

    out_ref[...] = jnp.dot(h_scr[...], w_scr[r], precision=_HIGH)


def _tc_project2(p1, bias1, coeff2, bases2):
    return pl.pallas_call(
        _tcB_body,
        grid=(NBLK, R),
        in_specs=[
            pl.BlockSpec((NC, BLK, H), lambda i, r: (0, i, 0)),
            pl.BlockSpec((H,), lambda i, r: (0,)),
            pl.BlockSpec((R, NB), lambda i, r: (0, 0)),
            pl.BlockSpec((NB, H, OUT), lambda i, r: (0, 0, 0)),
        ],
        out_specs=pl.BlockSpec((BLK, OUT), lambda i, r: (r * NBLK + i, 0)),
        out_shape=jax.ShapeDtypeStruct((R * N, OUT), jnp.float32),
        scratch_shapes=[pltpu.VMEM((BLK, H), jnp.float32),
                        pltpu.VMEM((R, H, OUT), jnp.float32)],
    )(p1, bias1, coeff2, bases2)


def _tcC_body(p_ref, b2_ref, out_ref):
    out_ref[...] = p_ref[0] + p_ref[1] + b2_ref[...][None, :]


def _tc_merge(p2, bias2):
    return pl.pallas_call(
        _tcC_body,
        grid=(NBLK,),
        in_specs=[
            pl.BlockSpec((NC, BLK, OUT), lambda i: (0, i, 0)),
            pl.BlockSpec((OUT,), lambda i: (0,)),
        ],
        out_specs=pl.BlockSpec((BLK, OUT), lambda i: (i, 0)),
        out_shape=jax.ShapeDtypeStruct((N, OUT), jnp.float32),
    )(p2, bias2)


def kernel(emb, W_sm, b_sm, coeff1, bases1, bias1, coeff2, bases2, bias2,
           norm, node_feats, edge_index, etypes):
    src = edge_index[0]
    dst = edge_index[1]
    nrm = norm.reshape(E)

    xw1 = _tc_project1(node_feats, emb, W_sm, b_sm, coeff1, bases1)
    p1 = _sc_aggregate(xw1, src, dst, etypes, nrm)
    xw2 = _tc_project2(p1, bias1, coeff2, bases2)
    p2 = _sc_aggregate(xw2, src, dst, etypes, nrm)
    return _tc_merge(p2, bias2)


# speedup vs baseline: 23.7733x; 1.1311x over previous
import functools

import jax
import jax.numpy as jnp
from jax import lax
from jax.experimental import pallas as pl
from jax.experimental.pallas import tpu as pltpu
from jax.experimental.pallas import tpu_sc as plsc

N = 10000
E = 320000
R = 8
NB = 8
H = 128
EDIM = 16
ML = 8
VOCAB = 4
OUT = 128

NC = 2
NS = 16
NW = NC * NS
EPW = E // NW
CHUNK = 80
NCHUNK = EPW // CHUNK
ZROWS = 624
ZREM = N - NS * ZROWS

BLK = 1000
NBLK = N // BLK

_HIGH = jax.lax.Precision.HIGHEST
_PROJ = jax.lax.Precision.DEFAULT


def _sc_body(table, src_h, dst_h, et_h, nrm_h, out_h,
             src_v, dst_v, et_v, nrm_v, gidx_v, rows_v, acc,
             isem0, isem1, gsem0, gsem1):
    c = lax.axis_index("c")
    s = lax.axis_index("s")
    wid = s * NC + c
    base = wid * EPW
    isems = (isem0, isem1)
    gsems = (gsem0, gsem1)

    zeros16 = jnp.zeros((16,), jnp.float32)

    @pl.loop(0, CHUNK)
    def _zero_rows(i):
        for f in range(H // 16):
            rows_v[0, i, pl.ds(f * 16, 16)] = zeros16

    zbase = s * ZROWS
    nfull = ZROWS // CHUNK
    tail = ZROWS - nfull * CHUNK

    @pl.loop(0, nfull)
    def _zero_acc(j):
        pltpu.sync_copy(rows_v.at[0], acc.at[pl.ds(zbase + j * CHUNK, CHUNK)])

    pltpu.sync_copy(rows_v.at[0, pl.ds(0, tail)],
                    acc.at[pl.ds(zbase + nfull * CHUNK, tail)])

    @pl.when(s == NS - 1)
    def _zero_rem():
        pltpu.sync_copy(rows_v.at[0, pl.ds(0, ZREM)],
                        acc.at[pl.ds(NS * ZROWS, ZREM)])

    plsc.subcore_barrier()


    def issue_idx(ch, b):
        o = base + ch * CHUNK
        pltpu.async_copy(src_h.at[pl.ds(o, CHUNK)], src_v.at[b], isems[b])
        pltpu.async_copy(et_h.at[pl.ds(o, CHUNK)], et_v.at[b], isems[b])
        pltpu.async_copy(dst_h.at[pl.ds(o, CHUNK)], dst_v.at[b], isems[b])
        pltpu.async_copy(nrm_h.at[pl.ds(o, CHUNK)], nrm_v.at[b], isems[b])

    def drain_idx(b):
        o = base
        for ref in (src_v, et_v, dst_v, nrm_v):
            pltpu.make_async_copy(src_h.at[pl.ds(o, CHUNK)], ref.at[b],
                                  isems[b]).wait()

    def compute_gidx(b):
        for g in range(CHUNK // 16):
            sl = pl.ds(g * 16, 16)
            gidx_v[b, sl] = et_v[b, sl] * N + src_v[b, sl]

    def issue_gather(b):
        pltpu.async_copy(table.at[gidx_v.at[b]], rows_v.at[b], gsems[b])

    def wait_gather(b):
        pltpu.make_async_copy(table.at[gidx_v.at[b]], rows_v.at[b],
                              gsems[b]).wait()

    def scale_scatter(b):
        @plsc.parallel_loop(0, CHUNK // 16, unroll=1)
        def _scale(g):
            g16 = nrm_v[b, pl.ds(g * 16, 16)]
            for j in range(16):
                nb = jnp.take_along_axis(g16, jnp.full((16,), j, jnp.int32),
                                         axis=0, mode="promise_in_bounds")
                e = g * 16 + j
                for f in range(H // 16):
                    sl = pl.ds(f * 16, 16)
                    rows_v[b, e, sl] = rows_v[b, e, sl] * nb

        pltpu.sync_copy(rows_v.at[b], acc.at[dst_v.at[b]], add=True)

    def half(c0, b):
        @pl.when(c0 + 1 < NCHUNK)
        def _():
            drain_idx(1 - b)
            compute_gidx(1 - b)
            issue_gather(1 - b)
        wait_gather(b)
        scale_scatter(b)
        @pl.when(c0 + 2 < NCHUNK)
        def _():
            issue_idx(c0 + 2, b)

    issue_idx(0, 0)
    drain_idx(0)
    compute_gidx(0)
    issue_gather(0)
    issue_idx(1, 1)

    @pl.loop(0, NCHUNK // 2)
    def _pair(i2):
        half(2 * i2, 0)
        half(2 * i2 + 1, 1)

    half(NCHUNK - 1, 0)

    plsc.subcore_barrier()

    pltpu.sync_copy(acc.at[pl.ds(zbase, ZROWS)],
                    out_h.at[c, pl.ds(zbase, ZROWS)])

    @pl.when(s == NS - 1)
    def _wb_rem():
        pltpu.sync_copy(acc.at[pl.ds(NS * ZROWS, ZREM)],
                        out_h.at[c, pl.ds(NS * ZROWS, ZREM)])


def _sc_aggregate(table, src, dst, etypes, nrm):
    mesh = plsc.VectorSubcoreMesh(core_axis_name="c", subcore_axis_name="s",
                                  num_cores=NC, num_subcores=NS)
    fn = pl.kernel(
        _sc_body,
        out_type=jax.ShapeDtypeStruct((NC, N, H), jnp.float32),
        mesh=mesh,
        scratch_types=[
            pltpu.VMEM((2, CHUNK), jnp.int32),
            pltpu.VMEM((2, CHUNK), jnp.int32),
            pltpu.VMEM((2, CHUNK), jnp.int32),
            pltpu.VMEM((2, CHUNK), jnp.float32),
            pltpu.VMEM((2, CHUNK), jnp.int32),
            pltpu.VMEM((2, CHUNK, H), jnp.float32),
            pltpu.VMEM_SHARED((N, H), jnp.float32),
            pltpu.SemaphoreType.DMA,
            pltpu.SemaphoreType.DMA,
            pltpu.SemaphoreType.DMA,
            pltpu.SemaphoreType.DMA,
        ],
        compiler_params=pltpu.CompilerParams(needs_layout_passes=False),
    )
    return fn(table, src, dst, etypes, nrm)


def _tcA_body(nf_ref, emb_ref, wsm_ref, bsm_ref, co_ref, ba_ref, out_ref,
              x_scr, w_scr):
    i = pl.program_id(0)
    r = pl.program_id(1)

    @pl.when(i == 0)
    def _():
        w_scr[r] = jnp.einsum("b,bio->io", co_ref[r], ba_ref[...],
                              precision=_HIGH)

    @pl.when(r == 0)
    def _():
        nf = nf_ref[...]
        col_m = lax.broadcasted_iota(jnp.int32, (BLK, ML * VOCAB), 1) // VOCAB
        col_v = lax.broadcasted_iota(jnp.int32, (BLK, ML * VOCAB), 1) % VOCAB
        nf_g = jnp.take_along_axis(nf, col_m, axis=1)
        oh = (nf_g == col_v).astype(jnp.float32)
        wsm3 = wsm_ref[...].reshape(ML, EDIM, H)
        m2 = jnp.einsum("vd,mdh->mvh", emb_ref[...], wsm3,
                        precision=_HIGH).reshape(ML * VOCAB, H)
        x_scr[...] = (jnp.dot(oh, m2, precision=_HIGH)
                      + bsm_ref[...][None, :])

    out_ref[...] = jnp.dot(x_scr[...], w_scr[r], precision=_PROJ)


def _tc_project1(node_feats, emb, W_sm, b_sm, coeff1, bases1):
    return pl.pallas_call(
        _tcA_body,
        grid=(NBLK, R),
        in_specs=[
            pl.BlockSpec((BLK, ML), lambda i, r: (i, 0)),
            pl.BlockSpec((VOCAB, EDIM), lambda i, r: (0, 0)),
            pl.BlockSpec((EDIM * ML, H), lambda i, r: (0, 0)),
            pl.BlockSpec((H,), lambda i, r: (0,)),
            pl.BlockSpec((R, NB), lambda i, r: (0, 0)),
            pl.BlockSpec((NB, H, H), lambda i, r: (0, 0, 0)),
        ],
        out_specs=pl.BlockSpec((BLK, H), lambda i, r: (r * NBLK + i, 0)),
        out_shape=jax.ShapeDtypeStruct((R * N, H), jnp.float32),
        scratch_shapes=[pltpu.VMEM((BLK, H), jnp.float32),
                        pltpu.VMEM((R, H, H), jnp.float32)],
    )(node_feats, emb, W_sm, b_sm, coeff1, bases1)


def _tcB_body(p_ref, b1_ref, co_ref, ba_ref, out_ref, h_scr, w_scr):
    i = pl.program_id(0)
    r = pl.program_id(1)

    @pl.when(i == 0)
    def _():
        w_scr[r] = jnp.einsum("b,bio->io", co_ref[r], ba_ref[...],
                              precision=_HIGH)

    @pl.when(r == 0)
    def _():
        h = p_ref[0] + p_ref[1] + b1_ref[...][None, :]
        h_scr[...] = jnp.maximum(h, 0.0)

    out_ref[...] = jnp.dot(h_scr[...], w_scr[r], precision=_PROJ)


def _tc_project2(p1, bias1, coeff2, bases2):
    return pl.pallas_call(
        _tcB_body,
        grid=(NBLK, R),
        in_specs=[
            pl.BlockSpec((NC, BLK, H), lambda i, r: (0, i, 0)),
            pl.BlockSpec((H,), lambda i, r: (0,)),
            pl.BlockSpec((R, NB), lambda i, r: (0, 0)),
            pl.BlockSpec((NB, H, OUT), lambda i, r: (0, 0, 0)),
        ],
        out_specs=pl.BlockSpec((BLK, OUT), lambda i, r: (r * NBLK + i, 0)),
        out_shape=jax.ShapeDtypeStruct((R * N, OUT), jnp.float32),
        scratch_shapes=[pltpu.VMEM((BLK, H), jnp.float32),
                        pltpu.VMEM((R, H, OUT), jnp.float32)],
    )(p1, bias1, coeff2, bases2)


def _tcC_body(p_ref, b2_ref, out_ref):
    out_ref[...] = p_ref[0] + p_ref[1] + b2_ref[...][None, :]


def _tc_merge(p2, bias2):
    return pl.pallas_call(
        _tcC_body,
        grid=(NBLK,),
        in_specs=[
            pl.BlockSpec((NC, BLK, OUT), lambda i: (0, i, 0)),
            pl.BlockSpec((OUT,), lambda i: (0,)),
        ],
        out_specs=pl.BlockSpec((BLK, OUT), lambda i: (i, 0)),
        out_shape=jax.ShapeDtypeStruct((N, OUT), jnp.float32),
    )(p2, bias2)


def kernel(emb, W_sm, b_sm, coeff1, bases1, bias1, coeff2, bases2, bias2,
           norm, node_feats, edge_index, etypes):
    src = edge_index[0]
    dst = edge_index[1]
    nrm = norm.reshape(E)

    xw1 = _tc_project1(node_feats, emb, W_sm, b_sm, coeff1, bases1)
    p1 = _sc_aggregate(xw1, src, dst, etypes, nrm)
    xw2 = _tc_project2(p1, bias1, coeff2, bases2)
    p2 = _sc_aggregate(xw2, src, dst, etypes, nrm)
    return _tc_merge(p2, bias2)


# speedup vs baseline: 27.2386x; 1.1458x over previous
import functools

import jax
import jax.numpy as jnp
from jax import lax
from jax.experimental import pallas as pl
from jax.experimental.pallas import tpu as pltpu
from jax.experimental.pallas import tpu_sc as plsc

N = 10000
E = 320000
R = 8
NB = 8
H = 128
EDIM = 16
ML = 8
VOCAB = 4
OUT = 128

NC = 2
NS = 16
NW = NC * NS
EPW = E // NW
CHUNK = 80
NCHUNK = EPW // CHUNK
ZROWS = 624
ZREM = N - NS * ZROWS

BLK = 1000
NBLK = N // BLK

_HIGH = jax.lax.Precision.HIGHEST
_PROJ = jax.lax.Precision.DEFAULT


SEB = 2000
NSEB = EPW // SEB


def _sc_body(table, src_h, dst_h, et_h, nrm_h, out_h,
             sbuf0, sbuf1, dbuf, ebuf0, ebuf1, nrm_v, gidx_v, rows_v, acc,
             isem, sesem0, sesem1, dsem0, dsem1, gsem0, gsem1):
    sbuf = (sbuf0, sbuf1)
    ebuf = (ebuf0, ebuf1)
    c = lax.axis_index("c")
    s = lax.axis_index("s")
    wid = s * NC + c
    gsems = (gsem0, gsem1)
    sesems = (sesem0, sesem1)
    dsems = (dsem0, dsem1)
    base = wid * EPW

    pltpu.async_copy(nrm_h.at[pl.ds(base, EPW)], nrm_v, isem)
    pltpu.async_copy(src_h.at[pl.ds(base, SEB)], sbuf[0], sesems[0])
    pltpu.async_copy(et_h.at[pl.ds(base, SEB)], ebuf[0], sesems[0])
    pltpu.async_copy(dst_h.at[wid, 0], dbuf.at[0], dsems[0])
    pltpu.async_copy(dst_h.at[wid, 1], dbuf.at[1], dsems[1])

    zeros16 = jnp.zeros((16,), jnp.float32)

    @pl.loop(0, CHUNK)
    def _zero_rows(i):
        for f in range(H // 16):
            rows_v[0, i, pl.ds(f * 16, 16)] = zeros16

    zbase = s * ZROWS
    nfull = ZROWS // CHUNK
    tail = ZROWS - nfull * CHUNK

    @pl.loop(0, nfull)
    def _zero_acc(j):
        pltpu.sync_copy(rows_v.at[0], acc.at[pl.ds(zbase + j * CHUNK, CHUNK)])

    pltpu.sync_copy(rows_v.at[0, pl.ds(0, tail)],
                    acc.at[pl.ds(zbase + nfull * CHUNK, tail)])

    @pl.when(s == NS - 1)
    def _zero_rem():
        pltpu.sync_copy(rows_v.at[0, pl.ds(0, ZREM)],
                        acc.at[pl.ds(NS * ZROWS, ZREM)])

    plsc.subcore_barrier()

    for k in range(NSEB):
        b = k % 2
        if k + 1 < NSEB:
            o = base + (k + 1) * SEB
            pltpu.async_copy(src_h.at[pl.ds(o, SEB)], sbuf[1 - b],
                             sesems[1 - b])
            pltpu.async_copy(et_h.at[pl.ds(o, SEB)], ebuf[1 - b],
                             sesems[1 - b])
        pltpu.make_async_copy(src_h.at[pl.ds(base, SEB)], sbuf[b],
                              sesems[b]).wait()
        pltpu.make_async_copy(et_h.at[pl.ds(base, SEB)], ebuf[b],
                              sesems[b]).wait()

        @pl.loop(0, SEB // 16)
        def _gidx(i):
            sl = pl.ds(i * 16, 16)
            gidx_v[pl.ds(k * SEB + i * 16, 16)] = (ebuf[b][sl] * N
                                                   + sbuf[b][sl])

    pltpu.make_async_copy(nrm_h.at[pl.ds(base, EPW)], nrm_v, isem).wait()


    def issue_gather(ch, b):
        pltpu.async_copy(table.at[gidx_v.at[pl.ds(ch * CHUNK, CHUNK)]],
                         rows_v.at[b], gsems[b])

    def wait_gather(ch, b):
        pltpu.make_async_copy(table.at[gidx_v.at[pl.ds(ch * CHUNK, CHUNK)]],
                              rows_v.at[b], gsems[b]).wait()

    def scale_scatter(ch, b):
        @plsc.parallel_loop(0, CHUNK // 16, unroll=1)
        def _scale(g):
            g16 = nrm_v[pl.ds(ch * CHUNK + g * 16, 16)]
            for j in range(16):
                nb = jnp.take_along_axis(g16, jnp.full((16,), j, jnp.int32),
                                         axis=0, mode="promise_in_bounds")
                e = g * 16 + j
                for f in range(H // 16):
                    sl = pl.ds(f * 16, 16)
                    rows_v[b, e, sl] = rows_v[b, e, sl] * nb

        pltpu.make_async_copy(dst_h.at[wid, ch], dbuf.at[b], dsems[b]).wait()
        pltpu.sync_copy(rows_v.at[b], acc.at[dbuf.at[b, 0]], add=True)

        @pl.when(ch + 2 < NCHUNK)
        def _():
            pltpu.async_copy(dst_h.at[wid, ch + 2], dbuf.at[b], dsems[b])

    def half(c0, b):
        @pl.when(c0 + 1 < NCHUNK)
        def _():
            issue_gather(c0 + 1, 1 - b)
        wait_gather(c0, b)
        scale_scatter(c0, b)

    issue_gather(0, 0)

    @pl.loop(0, NCHUNK // 2)
    def _pair(i2):
        half(2 * i2, 0)
        half(2 * i2 + 1, 1)

    half(NCHUNK - 1, 0)

    plsc.subcore_barrier()

    pltpu.sync_copy(acc.at[pl.ds(zbase, ZROWS)],
                    out_h.at[c, pl.ds(zbase, ZROWS)])

    @pl.when(s == NS - 1)
    def _wb_rem():
        pltpu.sync_copy(acc.at[pl.ds(NS * ZROWS, ZREM)],
                        out_h.at[c, pl.ds(NS * ZROWS, ZREM)])


def _sc_aggregate(table, src, dst, etypes, nrm):
    mesh = plsc.VectorSubcoreMesh(core_axis_name="c", subcore_axis_name="s",
                                  num_cores=NC, num_subcores=NS)
    fn = pl.kernel(
        _sc_body,
        out_type=jax.ShapeDtypeStruct((NC, N, H), jnp.float32),
        mesh=mesh,
        scratch_types=[
            pltpu.VMEM((SEB,), jnp.int32),
            pltpu.VMEM((SEB,), jnp.int32),
            pltpu.VMEM((2, 1, CHUNK), jnp.int32),
            pltpu.VMEM((SEB,), jnp.int32),
            pltpu.VMEM((SEB,), jnp.int32),
            pltpu.VMEM((EPW,), jnp.float32),
            pltpu.VMEM((EPW,), jnp.int32),
            pltpu.VMEM((2, CHUNK, H), jnp.float32),
            pltpu.VMEM_SHARED((N, H), jnp.float32),
            pltpu.SemaphoreType.DMA,
            pltpu.SemaphoreType.DMA,
            pltpu.SemaphoreType.DMA,
            pltpu.SemaphoreType.DMA,
            pltpu.SemaphoreType.DMA,
            pltpu.SemaphoreType.DMA,
            pltpu.SemaphoreType.DMA,
        ],
        compiler_params=pltpu.CompilerParams(needs_layout_passes=False),
    )
    return fn(table, src, dst.reshape(NW, NCHUNK, 1, CHUNK), etypes, nrm)


def _tcA_body(nf_ref, emb_ref, wsm_ref, bsm_ref, co_ref, ba_ref, out_ref,
              x_scr, w_scr):
    i = pl.program_id(0)
    r = pl.program_id(1)

    @pl.when(i == 0)
    def _():
        w_scr[r] = jnp.einsum("b,bio->io", co_ref[r], ba_ref[...],
                              precision=_HIGH)

    @pl.when(r == 0)
    def _():
        nf = nf_ref[...]
        col_m = lax.broadcasted_iota(jnp.int32, (BLK, ML * VOCAB), 1) // VOCAB
        col_v = lax.broadcasted_iota(jnp.int32, (BLK, ML * VOCAB), 1) % VOCAB
        nf_g = jnp.take_along_axis(nf, col_m, axis=1)
        oh = (nf_g == col_v).astype(jnp.float32)
        wsm3 = wsm_ref[...].reshape(ML, EDIM, H)
        m2 = jnp.einsum("vd,mdh->mvh", emb_ref[...], wsm3,
                        precision=_HIGH).reshape(ML * VOCAB, H)
        x_scr[...] = (jnp.dot(oh, m2, precision=_HIGH)
                      + bsm_ref[...][None, :])

    out_ref[...] = jnp.dot(x_scr[...], w_scr[r], precision=_PROJ)


def _tc_project1(node_feats, emb, W_sm, b_sm, coeff1, bases1):
    return pl.pallas_call(
        _tcA_body,
        grid=(NBLK, R),
        in_specs=[
            pl.BlockSpec((BLK, ML), lambda i, r: (i, 0)),
            pl.BlockSpec((VOCAB, EDIM), lambda i, r: (0, 0)),
            pl.BlockSpec((EDIM * ML, H), lambda i, r: (0, 0)),
            pl.BlockSpec((H,), lambda i, r: (0,)),
            pl.BlockSpec((R, NB), lambda i, r: (0, 0)),
            pl.BlockSpec((NB, H, H), lambda i, r: (0, 0, 0)),
        ],
        out_specs=pl.BlockSpec((BLK, H), lambda i, r: (r * NBLK + i, 0)),
        out_shape=jax.ShapeDtypeStruct((R * N, H), jnp.float32),
        scratch_shapes=[pltpu.VMEM((BLK, H), jnp.float32),
                        pltpu.VMEM((R, H, H), jnp.float32)],
    )(node_feats, emb, W_sm, b_sm, coeff1, bases1)


def _tcB_body(p_ref, b1_ref, co_ref, ba_ref, out_ref, h_scr, w_scr):
    i = pl.program_id(0)
    r = pl.program_id(1)

    @pl.when(i == 0)
    def _():
        w_scr[r] = jnp.einsum("b,bio->io", co_ref[r], ba_ref[...],
                              precision=_HIGH)

    @pl.when(r == 0)
    def _():
        h = p_ref[0] + p_ref[1] + b1_ref[...][None, :]
        h_scr[...] = jnp.maximum(h, 0.0)

    out_ref[...] = jnp.dot(h_scr[...], w_scr[r], precision=_PROJ)


def _tc_project2(p1, bias1, coeff2, bases2):
    return pl.pallas_call(
        _tcB_body,
        grid=(NBLK, R),
        in_specs=[
            pl.BlockSpec((NC, BLK, H), lambda i, r: (0, i, 0)),
            pl.BlockSpec((H,), lambda i, r: (0,)),
            pl.BlockSpec((R, NB), lambda i, r: (0, 0)),
            pl.BlockSpec((NB, H, OUT), lambda i, r: (0, 0, 0)),
        ],
        out_specs=pl.BlockSpec((BLK, OUT), lambda i, r: (r * NBLK + i, 0)),
        out_shape=jax.ShapeDtypeStruct((R * N, OUT), jnp.float32),
        scratch_shapes=[pltpu.VMEM((BLK, H), jnp.float32),
                        pltpu.VMEM((R, H, OUT), jnp.float32)],
    )(p1, bias1, coeff2, bases2)


def _tcC_body(p_ref, b2_ref, out_ref):
    out_ref[...] = p_ref[0] + p_ref[1] + b2_ref[...][None, :]


def _tc_merge(p2, bias2):
    return pl.pallas_call(
        _tcC_body,
        grid=(NBLK,),
        in_specs=[
            pl.BlockSpec((NC, BLK, OUT), lambda i: (0, i, 0)),
            pl.BlockSpec((OUT,), lambda i: (0,)),
        ],
        out_specs=pl.BlockSpec((BLK, OUT), lambda i: (i, 0)),
        out_shape=jax.ShapeDtypeStruct((N, OUT), jnp.float32),
    )(p2, bias2)


def kernel(emb, W_sm, b_sm, coeff1, bases1, bias1, coeff2, bases2, bias2,
           norm, node_feats, edge_index, etypes):
    src = edge_index[0]
    dst = edge_index[1]
    nrm = norm.reshape(E)

    xw1 = _tc_project1(node_feats, emb, W_sm, b_sm, coeff1, bases1)
    p1 = _sc_aggregate(xw1, src, dst, etypes, nrm)
    xw2 = _tc_project2(p1, bias1, coeff2, bases2)
    p2 = _sc_aggregate(xw2, src, dst, etypes, nrm)
    return _tc_merge(p2, bias2)
